# baseline (speedup 1.0000x reference)
import functools

import jax
import jax.numpy as jnp
from jax import lax
from jax.experimental import pallas as pl
from jax.experimental.pallas import tpu as pltpu
from jax.experimental.pallas import tpu_sc as plsc

MAX_LEN = 4096
HIDDEN_DIM = 1024
TC_BLOCK = 512

_MESH = plsc.ScalarSubcoreMesh(axis_name="c", num_cores=1)


@functools.partial(
    pl.kernel,
    out_type=jax.ShapeDtypeStruct((1, HIDDEN_DIM), jnp.float32),
    mesh=_MESH,
    scratch_types=[
        pltpu.SMEM((1,), jnp.int32),
    ],
)
def _sc_level_lookup(lev_hbm, table_hbm, row_hbm, lev_s):
    pltpu.sync_copy(lev_hbm, lev_s)
    levm1 = lev_s[0] - 1
    pltpu.sync_copy(table_hbm.at[pl.ds(levm1, 1)], row_hbm)


def _tc_broadcast_body(row_ref, out_ref):
    out_ref[...] = jnp.broadcast_to(row_ref[...], (TC_BLOCK, HIDDEN_DIM))


_tc_broadcast = pl.pallas_call(
    _tc_broadcast_body,
    grid=(MAX_LEN // TC_BLOCK,),
    in_specs=[pl.BlockSpec((1, HIDDEN_DIM), lambda i: (0, 0))],
    out_specs=pl.BlockSpec((TC_BLOCK, HIDDEN_DIM), lambda i: (i, 0)),
    out_shape=jax.ShapeDtypeStruct((MAX_LEN, HIDDEN_DIM), jnp.float32),
)


def kernel(x, lev, emb_table):
    lev_arr = jnp.asarray(lev, dtype=jnp.int32).reshape((1,))
    row = _sc_level_lookup(lev_arr, emb_table)
    out = _tc_broadcast(row)
    return out[None, : x.shape[1]]


# speedup vs baseline: 1.0281x; 1.0281x over previous
import functools

import jax
import jax.numpy as jnp
from jax import lax
from jax.experimental import pallas as pl
from jax.experimental.pallas import tpu as pltpu
from jax.experimental.pallas import tpu_sc as plsc

MAX_LEN = 4096
HIDDEN_DIM = 1024
TC_BLOCK = 1024

_MESH = plsc.ScalarSubcoreMesh(axis_name="c", num_cores=1)


@functools.partial(
    pl.kernel,
    out_type=jax.ShapeDtypeStruct((1, HIDDEN_DIM), jnp.float32),
    mesh=_MESH,
    scratch_types=[
        pltpu.SMEM((1,), jnp.int32),
    ],
)
def _sc_level_lookup(lev_hbm, table_hbm, row_hbm, lev_s):
    pltpu.sync_copy(lev_hbm, lev_s)
    levm1 = lev_s[0] - 1
    pltpu.sync_copy(table_hbm.at[pl.ds(levm1, 1)], row_hbm)


def _tc_broadcast_body(row_ref, out_ref, buf_ref, sem):
    buf_ref[...] = jnp.broadcast_to(row_ref[...], (TC_BLOCK, HIDDEN_DIM))
    descs = [
        pltpu.async_copy(
            buf_ref, out_ref.at[pl.ds(i * TC_BLOCK, TC_BLOCK)], sem
        )
        for i in range(MAX_LEN // TC_BLOCK)
    ]
    for d in descs:
        d.wait()


_tc_broadcast = pl.pallas_call(
    _tc_broadcast_body,
    in_specs=[pl.BlockSpec((1, HIDDEN_DIM), lambda: (0, 0))],
    out_specs=pl.BlockSpec(memory_space=pl.ANY),
    out_shape=jax.ShapeDtypeStruct((MAX_LEN, HIDDEN_DIM), jnp.float32),
    scratch_shapes=[
        pltpu.VMEM((TC_BLOCK, HIDDEN_DIM), jnp.float32),
        pltpu.SemaphoreType.DMA,
    ],
)


def kernel(x, lev, emb_table):
    lev_arr = jnp.asarray(lev, dtype=jnp.int32).reshape((1,))
    row = _sc_level_lookup(lev_arr, emb_table)
    out = _tc_broadcast(row)
    return out[None, : x.shape[1]]


# speedup vs baseline: 1.0379x; 1.0095x over previous
import functools

import jax
import jax.numpy as jnp
from jax import lax
from jax.experimental import pallas as pl
from jax.experimental.pallas import tpu as pltpu
from jax.experimental.pallas import tpu_sc as plsc

MAX_LEN = 4096
HIDDEN_DIM = 1024
TC_BLOCK = 256

_MESH = plsc.ScalarSubcoreMesh(axis_name="c", num_cores=1)


@functools.partial(
    pl.kernel,
    out_type=jax.ShapeDtypeStruct((1, HIDDEN_DIM), jnp.float32),
    mesh=_MESH,
    scratch_types=[
        pltpu.SMEM((1,), jnp.int32),
    ],
)
def _sc_level_lookup(lev_hbm, table_hbm, row_hbm, lev_s):
    pltpu.sync_copy(lev_hbm, lev_s)
    levm1 = lev_s[0] - 1
    pltpu.sync_copy(table_hbm.at[pl.ds(levm1, 1)], row_hbm)


def _tc_broadcast_body(row_ref, out_ref, buf_ref, sem):
    buf_ref[...] = jnp.broadcast_to(row_ref[...], (TC_BLOCK, HIDDEN_DIM))
    descs = [
        pltpu.async_copy(
            buf_ref, out_ref.at[pl.ds(i * TC_BLOCK, TC_BLOCK)], sem
        )
        for i in range(MAX_LEN // TC_BLOCK)
    ]
    for d in descs:
        d.wait()


_tc_broadcast = pl.pallas_call(
    _tc_broadcast_body,
    in_specs=[pl.BlockSpec((1, HIDDEN_DIM), lambda: (0, 0))],
    out_specs=pl.BlockSpec(memory_space=pl.ANY),
    out_shape=jax.ShapeDtypeStruct((MAX_LEN, HIDDEN_DIM), jnp.float32),
    scratch_shapes=[
        pltpu.VMEM((TC_BLOCK, HIDDEN_DIM), jnp.float32),
        pltpu.SemaphoreType.DMA,
    ],
)


def kernel(x, lev, emb_table):
    lev_arr = jnp.asarray(lev, dtype=jnp.int32).reshape((1,))
    row = _sc_level_lookup(lev_arr, emb_table)
    out = _tc_broadcast(row)
    return out[None, : x.shape[1]]
